# baseline (speedup 1.0000x reference)
import functools

import jax
import jax.numpy as jnp
from jax import lax
from jax.experimental import pallas as pl
from jax.experimental.pallas import tpu as pltpu
from jax.experimental.pallas import tpu_sc as plsc

_N_NODES = 100000
_N_GEN = 8
_C = 16
_M = 256
_N_GRAPHS = 512

_NC = 2
_NS = 16
_NW = _NC * _NS
_CHUNK = 128
_CHUNKS_PER_W = 25
_NPW = _CHUNK * _CHUNKS_PER_W
_NPAD = _NW * _NPW
_GP = 16
_TROWS = 512


def _table_body(b_ref, pi_ref, t_ref):
    pi = pi_ref[...]
    pim = jnp.max(pi, axis=0, keepdims=True)
    pe = jnp.exp(pi - pim)
    log_pi = pi - pim - jnp.log(jnp.sum(pe, axis=0, keepdims=True))
    rows = []
    for g in range(_N_GEN):
        bg = b_ref[g]
        bm = jnp.max(bg, axis=1, keepdims=True)
        be = jnp.exp(bg - bm)
        log_b = bg - bm - jnp.log(jnp.sum(be, axis=1, keepdims=True))
        ln = log_b + log_pi[:, g:g + 1]
        mx = jnp.max(ln, axis=0, keepdims=True)
        p = jnp.exp(ln - mx)
        p = p / jnp.sum(p, axis=0, keepdims=True)
        rows.append(jnp.sum(p * ln, axis=0, keepdims=True))
    t_gm = jnp.concatenate(rows, axis=0)
    t_mg = t_gm.T
    t_ref[...] = jnp.zeros((_TROWS, _GP), jnp.float32)
    t_ref[0:_M, 0:_N_GEN] = t_mg


def _combine_body(p_ref, o_ref):
    o_ref[...] = -(p_ref[0, :, :_N_GEN] + p_ref[1, :, :_N_GEN])


def _sc_body(t_hbm, xi_hbm, bi_hbm, out_hbm, xv, bv, rows, tmp, acc, semi, semg, sems):
    cid = lax.axis_index("c")
    sid = lax.axis_index("s")
    wid = sid * _NC + cid

    hx = pltpu.async_copy(xi_hbm.at[wid], xv, semi)
    hb = pltpu.async_copy(bi_hbm.at[wid], bv, semi)

    for i in range(_N_GRAPHS // _NS):
        tmp[i, :] = jnp.zeros((_GP,), jnp.float32)
    pltpu.sync_copy(tmp, acc.at[pl.ds(sid * (_N_GRAPHS // _NS), _N_GRAPHS // _NS)])

    hx.wait()
    hb.wait()

    gh = [
        pltpu.async_copy(t_hbm.at[xv.at[j]], rows.at[j], semg)
        for j in range(_CHUNKS_PER_W)
    ]
    plsc.subcore_barrier()
    for h in gh:
        h.wait()
    sh = [
        pltpu.async_copy(rows.at[j], acc.at[bv.at[j]], sems, add=True)
        for j in range(_CHUNKS_PER_W)
    ]
    for h in sh:
        h.wait()

    plsc.subcore_barrier()

    pltpu.sync_copy(acc.at[pl.ds(sid * (_N_GRAPHS // _NS), _N_GRAPHS // _NS)], tmp)
    pltpu.sync_copy(tmp, out_hbm.at[cid, pl.ds(sid * (_N_GRAPHS // _NS), _N_GRAPHS // _NS)])


@functools.lru_cache(maxsize=1)
def _sc_call():
    return pl.kernel(
        _sc_body,
        out_type=jax.ShapeDtypeStruct((_NC, _N_GRAPHS, _GP), jnp.float32),
        mesh=plsc.VectorSubcoreMesh(
            core_axis_name="c", subcore_axis_name="s",
            num_cores=_NC, num_subcores=_NS),
        scratch_types=[
            pltpu.VMEM((_CHUNKS_PER_W, _CHUNK), jnp.int32),
            pltpu.VMEM((_CHUNKS_PER_W, _CHUNK), jnp.int32),
            pltpu.VMEM((_CHUNKS_PER_W, _CHUNK, _GP), jnp.float32),
            pltpu.VMEM((_N_GRAPHS // _NS, _GP), jnp.float32),
            pltpu.VMEM_SHARED((_N_GRAPHS, _GP), jnp.float32),
            pltpu.SemaphoreType.DMA,
            pltpu.SemaphoreType.DMA,
            pltpu.SemaphoreType.DMA,
        ],
        compiler_params=pltpu.CompilerParams(use_tc_tiling_on_sc=False),
    )


@jax.jit
def kernel(x, edge_index, batch, B, Pi):
    del edge_index

    t_pad = pl.pallas_call(
        _table_body,
        out_shape=jax.ShapeDtypeStruct((_TROWS, _GP), jnp.float32),
    )(jnp.transpose(B, (2, 0, 1)), Pi)

    pad_x = jnp.full((_NPAD - _N_NODES,), _M, jnp.int32)
    pad_b = jnp.zeros((_NPAD - _N_NODES,), jnp.int32)
    xi = jnp.concatenate([x.astype(jnp.int32), pad_x]).reshape(_NW, _CHUNKS_PER_W, _CHUNK)
    bi = jnp.concatenate([batch.astype(jnp.int32), pad_b]).reshape(_NW, _CHUNKS_PER_W, _CHUNK)

    partials = _sc_call()(t_pad, xi, bi)

    out = pl.pallas_call(
        _combine_body,
        out_shape=jax.ShapeDtypeStruct((_N_GRAPHS, _N_GEN), jnp.float32),
    )(partials)
    return out.reshape(_N_GRAPHS, 1, _N_GEN)


# speedup vs baseline: 1.6123x; 1.6123x over previous
import functools

import jax
import jax.numpy as jnp
from jax import lax
from jax.experimental import pallas as pl
from jax.experimental.pallas import tpu as pltpu
from jax.experimental.pallas import tpu_sc as plsc

_N_NODES = 100000
_N_GEN = 8
_C = 16
_M = 256
_N_GRAPHS = 512

_NC = 2
_NS = 16
_NW = _NC * _NS
_CHUNK = 128
_CHUNKS_PER_W = 25
_NPW = _CHUNK * _CHUNKS_PER_W
_NPAD = _NW * _NPW
_GP = 16
_TROWS = 512


def _table_body(b_ref, pi_ref, t_ref):
    pi = pi_ref[...]
    pim = jnp.max(pi, axis=0, keepdims=True)
    pe = jnp.exp(pi - pim)
    log_pi = pi - pim - jnp.log(jnp.sum(pe, axis=0, keepdims=True))
    rows = []
    for g in range(_N_GEN):
        bg = b_ref[g]
        bm = jnp.max(bg, axis=1, keepdims=True)
        be = jnp.exp(bg - bm)
        log_b = bg - bm - jnp.log(jnp.sum(be, axis=1, keepdims=True))
        ln = log_b + log_pi[:, g:g + 1]
        mx = jnp.max(ln, axis=0, keepdims=True)
        p = jnp.exp(ln - mx)
        p = p / jnp.sum(p, axis=0, keepdims=True)
        rows.append(jnp.sum(p * ln, axis=0, keepdims=True))
    t_gm = jnp.concatenate(rows, axis=0)
    t_mg = t_gm.T
    t_ref[...] = jnp.zeros((_TROWS, _GP), jnp.float32)
    t_ref[0:_M, 0:_N_GEN] = t_mg


def _combine_body(p_ref, o_ref):
    o_ref[...] = -(p_ref[0, :, :_N_GEN] + p_ref[1, :, :_N_GEN])


def _sc_body(t_hbm, xi_hbm, bi_hbm, out_hbm, xv, bv, rows, tmp, acc, tsh, semi, semg, sems):
    cid = lax.axis_index("c")
    sid = lax.axis_index("s")
    wid = sid * _NC + cid

    hx = pltpu.async_copy(xi_hbm.at[wid], xv, semi)
    hb = pltpu.async_copy(bi_hbm.at[wid], bv, semi)

    @pl.when(sid == 0)
    def _():
        pltpu.sync_copy(t_hbm, tsh)

    for i in range(_N_GRAPHS // _NS):
        tmp[i, :] = jnp.zeros((_GP,), jnp.float32)
    pltpu.sync_copy(tmp, acc.at[pl.ds(sid * (_N_GRAPHS // _NS), _N_GRAPHS // _NS)])

    hx.wait()
    hb.wait()
    plsc.subcore_barrier()

    gh = [
        pltpu.async_copy(tsh.at[xv.at[j]], rows.at[j], semg)
        for j in range(_CHUNKS_PER_W)
    ]
    for h in gh:
        h.wait()
    sh = [
        pltpu.async_copy(rows.at[j], acc.at[bv.at[j]], sems, add=True)
        for j in range(_CHUNKS_PER_W)
    ]
    for h in sh:
        h.wait()

    plsc.subcore_barrier()

    pltpu.sync_copy(acc.at[pl.ds(sid * (_N_GRAPHS // _NS), _N_GRAPHS // _NS)], tmp)
    pltpu.sync_copy(tmp, out_hbm.at[cid, pl.ds(sid * (_N_GRAPHS // _NS), _N_GRAPHS // _NS)])


@functools.lru_cache(maxsize=1)
def _sc_call():
    return pl.kernel(
        _sc_body,
        out_type=jax.ShapeDtypeStruct((_NC, _N_GRAPHS, _GP), jnp.float32),
        mesh=plsc.VectorSubcoreMesh(
            core_axis_name="c", subcore_axis_name="s",
            num_cores=_NC, num_subcores=_NS),
        scratch_types=[
            pltpu.VMEM((_CHUNKS_PER_W, _CHUNK), jnp.int32),
            pltpu.VMEM((_CHUNKS_PER_W, _CHUNK), jnp.int32),
            pltpu.VMEM((_CHUNKS_PER_W, _CHUNK, _GP), jnp.float32),
            pltpu.VMEM((_N_GRAPHS // _NS, _GP), jnp.float32),
            pltpu.VMEM_SHARED((_N_GRAPHS, _GP), jnp.float32),
            pltpu.VMEM_SHARED((_TROWS, _GP), jnp.float32),
            pltpu.SemaphoreType.DMA,
            pltpu.SemaphoreType.DMA,
            pltpu.SemaphoreType.DMA,
        ],
        compiler_params=pltpu.CompilerParams(use_tc_tiling_on_sc=False),
    )


@jax.jit
def kernel(x, edge_index, batch, B, Pi):
    del edge_index

    t_pad = pl.pallas_call(
        _table_body,
        out_shape=jax.ShapeDtypeStruct((_TROWS, _GP), jnp.float32),
    )(jnp.transpose(B, (2, 0, 1)), Pi)

    pad_x = jnp.full((_NPAD - _N_NODES,), _M, jnp.int32)
    pad_b = jnp.zeros((_NPAD - _N_NODES,), jnp.int32)
    xi = jnp.concatenate([x.astype(jnp.int32), pad_x]).reshape(_NW, _CHUNKS_PER_W, _CHUNK)
    bi = jnp.concatenate([batch.astype(jnp.int32), pad_b]).reshape(_NW, _CHUNKS_PER_W, _CHUNK)

    partials = _sc_call()(t_pad, xi, bi)

    out = pl.pallas_call(
        _combine_body,
        out_shape=jax.ShapeDtypeStruct((_N_GRAPHS, _N_GEN), jnp.float32),
    )(partials)
    return out.reshape(_N_GRAPHS, 1, _N_GEN)
